# baseline (speedup 1.0000x reference)
import jax
import jax.numpy as jnp
from jax import lax
from jax.experimental import pallas as pl
from jax.experimental.pallas import tpu as pltpu
from jax.experimental.pallas import tpu_sc as plsc

_BATCH = 16384
_D = 64
_NC = 2
_NS = 16
_NW = _NC * _NS
_NLOC = 1000000
_RPC = 512
_NCHUNKS = (_NLOC + _RPC - 1) // _RPC
_TAILK = _NLOC // _RPC
_TAIL0 = _TAILK * _RPC
_MAXT = (_NCHUNKS - 1) // _NW + 1


def _body(lidx_hbm, aidx_hbm, wt_hbm, at_hbm, tail_hbm, out_hbm,
          lidx_v, aidx_v, mylist_v, chunk_v, atab_v, comb_v, bidx_v,
          hist_s, start_s, gsem0, gsem1, ssem0, ssem1):
    wid = lax.axis_index("s") * _NC + lax.axis_index("c")
    sub = lax.iota(jnp.int32, 16)

    for t0 in range(2):
        pltpu.async_copy(wt_hbm.at[:, pl.ds((wid + t0 * _NW) * _RPC, _RPC)],
                         chunk_v.at[t0], (gsem0, gsem1)[t0])

    pltpu.sync_copy(lidx_hbm, lidx_v.at[pl.ds(0, _BATCH)])
    pltpu.sync_copy(aidx_hbm, aidx_v.at[pl.ds(0, _BATCH)])
    pltpu.sync_copy(at_hbm, atab_v)

    def scan_step(v, cnt):
        r = lidx_v[pl.ds(v * 16, 16)]
        k = lax.shift_right_logical(r, 9)
        mine = (k & (_NW - 1)) == wid
        t = lax.shift_right_logical(r, 14)
        x = r & (_RPC - 1)
        b = v * 16 + sub
        packed = (t << 23) | (x << 14) | b
        plsc.store_compressed(mylist_v.at[pl.ds(cnt, 16)], packed, mask=mine)
        return cnt + plsc.all_reduce_population_count(mine)[0]

    n = lax.fori_loop(0, _BATCH // 16, scan_step, jnp.int32(0))

    def zero_step(t, c):
        hist_s[t] = jnp.int32(0)
        return c
    lax.fori_loop(0, _MAXT + 1, zero_step, 0)

    nfull = lax.shift_right_logical(n, 4)
    ntail = n & 15

    def hist_full(v, c):
        pv = mylist_v[pl.ds(v * 16, 16)]
        for j in range(16):
            t = lax.shift_right_logical(pv[j], 23)
            hist_s[t] = hist_s[t] + 1
        return c
    lax.fori_loop(0, nfull, hist_full, 0)

    @pl.when(ntail > 0)
    def _hist_tail():
        pv = mylist_v[pl.ds(nfull * 16, 16)]
        for j in range(16):
            @pl.when(j < ntail)
            def _do():
                t = lax.shift_right_logical(pv[j], 23)
                hist_s[t] = hist_s[t] + 1

    def pfx_step(t, acc):
        c = hist_s[t]
        start_s[t] = acc
        hist_s[t] = acc
        return acc + c
    total = lax.fori_loop(0, _MAXT + 1, pfx_step, jnp.int32(0))
    start_s[_MAXT + 1] = total

    lane0 = sub == 0

    def sort_full(v, c):
        pv = mylist_v[pl.ds(v * 16, 16)]
        for j in range(16):
            t = lax.shift_right_logical(pv[j], 23)
            p = hist_s[t]
            hist_s[t] = p + 1
            idx = jnp.broadcast_to(p, (16,))
            plsc.store_scatter(lidx_v, [idx],
                               jnp.broadcast_to(pv[j], (16,)), mask=lane0)
        return c
    lax.fori_loop(0, nfull, sort_full, 0)

    @pl.when(ntail > 0)
    def _sort_tail():
        pv = mylist_v[pl.ds(nfull * 16, 16)]
        for j in range(16):
            @pl.when(j < ntail)
            def _do():
                t = lax.shift_right_logical(pv[j], 23)
                p = hist_s[t]
                hist_s[t] = p + 1
                idx = jnp.broadcast_to(p, (16,))
                plsc.store_scatter(lidx_v, [idx],
                                   jnp.broadcast_to(pv[j], (16,)), mask=lane0)

    nt = lax.div(jnp.int32(_NCHUNKS) - 1 - wid, _NW) + 1

    def m_of(t):
        return start_s[t + 1] - start_s[t]

    def fire(t, slot, gsem):
        k = wid + t * _NW

        @pl.when(m_of(t) > 0)
        def _f():
            @pl.when(k < _TAILK)
            def _a():
                pltpu.async_copy(wt_hbm.at[:, pl.ds(k * _RPC, _RPC)],
                                 chunk_v.at[slot], gsem)

            @pl.when(k == _TAILK)
            def _b():
                pltpu.async_copy(tail_hbm, chunk_v.at[slot], gsem)

    def wait_fetch(t, slot, gsem):
        @pl.when((m_of(t) > 0) | (t < 2))
        def _w():
            pltpu.make_async_copy(tail_hbm, chunk_v.at[slot], gsem).wait()

    def process(t, cslot, carry):
        s0 = start_s[t]
        m = m_of(t)
        ng = lax.div(m + 15, 16)

        def group(g, gpar, gg, ssem):
            pv = lidx_v[pl.ds(s0 + g * 16, 16)]
            rem = m - g * 16
            bvec = pv & jnp.int32(16383)
            bvec = jnp.where(sub < rem, bvec, jnp.int32(_BATCH))

            @pl.when(gg >= 1)
            def _wprev():
                pltpu.make_async_copy(comb_v.at[gpar],
                                      out_hbm.at[bidx_v.at[gpar]], ssem).wait()

            for j in range(16):
                @pl.when(j < rem)
                def _fill():
                    x = lax.shift_right_logical(pv[j], 14) & (_RPC - 1)
                    a = aidx_v[pl.ds(pv[j] & 16383, 16)][0]
                    for c in range(_D // 16):
                        d16 = sub + c * 16
                        ag = plsc.load_gather(
                            atab_v, [d16, jnp.broadcast_to(a, (16,))])
                        comb_v[gpar, j, pl.ds(c * 16, 16)] = ag
                        lg = plsc.load_gather(
                            chunk_v.at[cslot],
                            [d16, jnp.broadcast_to(x, (16,))])
                        comb_v[gpar, j, pl.ds(_D + c * 16, 16)] = lg

            bidx_v[gpar, :] = bvec
            pltpu.async_copy(comb_v.at[gpar], out_hbm.at[bidx_v.at[gpar]], ssem)
            return gg + 1

        def gpair(gp, carry):
            g0 = gp * 2
            g1 = gp * 2 + 1
            if cslot == 0:
                gg0, gg1 = carry
                gg0 = lax.cond(g0 < ng, lambda gg: group(g0, 0, gg, ssem0),
                               lambda gg: gg, gg0)
                gg1 = lax.cond(g1 < ng, lambda gg: group(g1, 1, gg, ssem1),
                               lambda gg: gg, gg1)
                return (gg0, gg1)
            else:
                gg0, gg1 = carry
                gg1 = lax.cond(g0 < ng, lambda gg: group(g0, 1, gg, ssem1),
                               lambda gg: gg, gg1)
                gg0 = lax.cond(g1 < ng, lambda gg: group(g1, 0, gg, ssem0),
                               lambda gg: gg, gg0)
                return (gg0, gg1)

        return lax.fori_loop(0, lax.div(ng + 1, 2), gpair, carry)

    def chunk_pair(tp, carry):
        t0 = tp * 2
        t1 = tp * 2 + 1

        def do0(c):
            wait_fetch(t0, 0, gsem0)
            c = process(t0, 0, c)

            @pl.when(t0 + 2 < nt)
            def _f2():
                fire(t0 + 2, 0, gsem0)
            return c
        carry = lax.cond(t0 < nt, do0, lambda c: c, carry)

        def do1(c):
            wait_fetch(t1, 1, gsem1)
            c = process(t1, 1, c)

            @pl.when(t1 + 2 < nt)
            def _f3():
                fire(t1 + 2, 1, gsem1)
            return c
        carry = lax.cond(t1 < nt, do1, lambda c: c, carry)
        return carry

    gg0, gg1 = lax.fori_loop(0, lax.div(nt + 1, 2), chunk_pair,
                             (jnp.int32(0), jnp.int32(0)))

    @pl.when(gg0 >= 1)
    def _d0():
        pltpu.make_async_copy(comb_v.at[0], out_hbm.at[bidx_v.at[0]], ssem0).wait()

    @pl.when(gg1 >= 1)
    def _d1():
        pltpu.make_async_copy(comb_v.at[1], out_hbm.at[bidx_v.at[1]], ssem1).wait()


def kernel(location_idx, age_idx, W_location, W_age):
    lidx = location_idx.astype(jnp.int32)
    aidx = age_idx.astype(jnp.int32)
    WT = W_location.T
    AT = W_age.T
    tailT = jnp.pad(W_location[_TAIL0:].T,
                    ((0, 0), (0, _RPC - (_NLOC - _TAIL0))))

    mesh = plsc.VectorSubcoreMesh(core_axis_name="c", subcore_axis_name="s")
    run = pl.kernel(
        _body,
        out_type=jax.ShapeDtypeStruct((_BATCH + 1, 2 * _D), jnp.float32),
        mesh=mesh,
        scratch_types=[
            pltpu.VMEM((_BATCH + 16,), jnp.int32),
            pltpu.VMEM((_BATCH + 16,), jnp.int32),
            pltpu.VMEM((_BATCH + 16,), jnp.int32),
            pltpu.VMEM((2, _D, _RPC), jnp.float32),
            pltpu.VMEM((_D, 100), jnp.float32),
            pltpu.VMEM((2, 16, 2 * _D), jnp.float32),
            pltpu.VMEM((2, 16), jnp.int32),
            pltpu.SMEM((_MAXT + 2,), jnp.int32),
            pltpu.SMEM((_MAXT + 2,), jnp.int32),
            pltpu.SemaphoreType.DMA,
            pltpu.SemaphoreType.DMA,
            pltpu.SemaphoreType.DMA,
            pltpu.SemaphoreType.DMA,
        ],
        compiler_params=pltpu.CompilerParams(needs_layout_passes=False),
    )
    return run(lidx, aidx, WT, AT, tailT)[:_BATCH]


# speedup vs baseline: 5.7164x; 5.7164x over previous
import jax
import jax.numpy as jnp
from jax import lax
from jax.experimental import pallas as pl
from jax.experimental.pallas import tpu as pltpu
from jax.experimental.pallas import tpu_sc as plsc

_BATCH = 16384
_D = 64
_NC = 2
_NS = 16
_NW = _NC * _NS
_NLOC = 1000000
_RPC = 512
_NCHUNKS = (_NLOC + _RPC - 1) // _RPC
_TAILK = _NLOC // _RPC
_TAIL0 = _TAILK * _RPC
_MAXT = (_NCHUNKS - 1) // _NW + 1


def _body(lidx_hbm, aidx_hbm, wt_hbm, at_hbm, tail_hbm, out_hbm,
          lidx_v, aidx_v, mylist_v, chunk_v, atab_v, comb_v, bidx_v,
          hist_s, start_s, gsem0, gsem1, ssem0, ssem1):
    wid = lax.axis_index("s") * _NC + lax.axis_index("c")
    sub = lax.iota(jnp.int32, 16)

    for t0 in range(2):
        pltpu.async_copy(wt_hbm.at[:, pl.ds((wid + t0 * _NW) * _RPC, _RPC)],
                         chunk_v.at[t0], (gsem0, gsem1)[t0])

    pltpu.sync_copy(lidx_hbm, lidx_v.at[pl.ds(0, _BATCH)])
    pltpu.sync_copy(aidx_hbm, aidx_v.at[pl.ds(0, _BATCH)])
    pltpu.sync_copy(at_hbm, atab_v)

    def scan_step(v, cnt):
        r = lidx_v[pl.ds(v * 16, 16)]
        k = lax.shift_right_logical(r, 9)
        mine = (k & (_NW - 1)) == wid
        t = lax.shift_right_logical(r, 14)
        x = r & (_RPC - 1)
        b = v * 16 + sub
        packed = (t << 23) | (x << 14) | b
        plsc.store_compressed(mylist_v.at[pl.ds(cnt, 16)], packed, mask=mine)
        return cnt + plsc.all_reduce_population_count(mine)[0]

    n = lax.fori_loop(0, _BATCH // 16, scan_step, jnp.int32(0))

    def zero_step(t, c):
        hist_s[t] = jnp.int32(0)
        return c
    lax.fori_loop(0, _MAXT + 1, zero_step, 0)

    nfull = lax.shift_right_logical(n, 4)
    ntail = n & 15

    def hist_full(v, c):
        pv = mylist_v[pl.ds(v * 16, 16)]
        for j in range(16):
            t = lax.shift_right_logical(pv[j], 23)
            hist_s[t] = hist_s[t] + 1
        return c
    lax.fori_loop(0, nfull, hist_full, 0)

    @pl.when(ntail > 0)
    def _hist_tail():
        pv = mylist_v[pl.ds(nfull * 16, 16)]
        for j in range(16):
            @pl.when(j < ntail)
            def _do():
                t = lax.shift_right_logical(pv[j], 23)
                hist_s[t] = hist_s[t] + 1

    def pfx_step(t, acc):
        c = hist_s[t]
        start_s[t] = acc
        hist_s[t] = acc
        return acc + c
    total = lax.fori_loop(0, _MAXT + 1, pfx_step, jnp.int32(0))
    start_s[_MAXT + 1] = total

    lane0 = sub == 0

    def sort_full(v, c):
        pv = mylist_v[pl.ds(v * 16, 16)]
        for j in range(16):
            t = lax.shift_right_logical(pv[j], 23)
            p = hist_s[t]
            hist_s[t] = p + 1
            idx = jnp.broadcast_to(p, (16,))
            plsc.store_scatter(lidx_v, [idx],
                               jnp.broadcast_to(pv[j], (16,)), mask=lane0)
        return c
    lax.fori_loop(0, nfull, sort_full, 0)

    @pl.when(ntail > 0)
    def _sort_tail():
        pv = mylist_v[pl.ds(nfull * 16, 16)]
        for j in range(16):
            @pl.when(j < ntail)
            def _do():
                t = lax.shift_right_logical(pv[j], 23)
                p = hist_s[t]
                hist_s[t] = p + 1
                idx = jnp.broadcast_to(p, (16,))
                plsc.store_scatter(lidx_v, [idx],
                                   jnp.broadcast_to(pv[j], (16,)), mask=lane0)

    nt = lax.div(jnp.int32(_NCHUNKS) - 1 - wid, _NW) + 1

    def m_of(t):
        return start_s[t + 1] - start_s[t]

    def fire(t, slot, gsem):
        k = wid + t * _NW

        @pl.when(m_of(t) > 0)
        def _f():
            @pl.when(k < _TAILK)
            def _a():
                pltpu.async_copy(wt_hbm.at[:, pl.ds(k * _RPC, _RPC)],
                                 chunk_v.at[slot], gsem)

            @pl.when(k == _TAILK)
            def _b():
                pltpu.async_copy(tail_hbm, chunk_v.at[slot], gsem)

    def wait_fetch(t, slot, gsem):
        @pl.when((m_of(t) > 0) | (t < 2))
        def _w():
            pltpu.make_async_copy(tail_hbm, chunk_v.at[slot], gsem).wait()

    def process(t, cslot, carry):
        s0 = start_s[t]
        m = m_of(t)
        ng = lax.div(m + 15, 16)

        def group(g, gpar, gg, ssem):
            pv = lidx_v[pl.ds(s0 + g * 16, 16)]
            rem = m - g * 16
            bvec = pv & jnp.int32(16383)
            bvec = jnp.where(sub < rem, bvec, jnp.broadcast_to(bvec[0], (16,)))

            @pl.when(gg >= 1)
            def _wprev():
                pltpu.make_async_copy(comb_v.at[gpar],
                                      out_hbm.at[bidx_v.at[gpar]], ssem).wait()

            for j in range(16):
                @pl.when(j < rem)
                def _fill():
                    x = lax.shift_right_logical(pv[j], 14) & (_RPC - 1)
                    a = aidx_v[pl.ds(pv[j] & 16383, 16)][0]
                    for c in range(_D // 16):
                        d16 = sub + c * 16
                        ag = plsc.load_gather(
                            atab_v, [d16, jnp.broadcast_to(a, (16,))])
                        comb_v[gpar, j, pl.ds(c * 16, 16)] = ag
                        lg = plsc.load_gather(
                            chunk_v.at[cslot],
                            [d16, jnp.broadcast_to(x, (16,))])
                        comb_v[gpar, j, pl.ds(_D + c * 16, 16)] = lg

                @pl.when(j >= rem)
                def _pad():
                    for c in range(2 * _D // 16):
                        comb_v[gpar, j, pl.ds(c * 16, 16)] = \
                            comb_v[gpar, 0, pl.ds(c * 16, 16)]

            bidx_v[gpar, :] = bvec
            pltpu.async_copy(comb_v.at[gpar], out_hbm.at[bidx_v.at[gpar]], ssem)
            return gg + 1

        def gpair(gp, carry):
            g0 = gp * 2
            g1 = gp * 2 + 1
            if cslot == 0:
                gg0, gg1 = carry
                gg0 = lax.cond(g0 < ng, lambda gg: group(g0, 0, gg, ssem0),
                               lambda gg: gg, gg0)
                gg1 = lax.cond(g1 < ng, lambda gg: group(g1, 1, gg, ssem1),
                               lambda gg: gg, gg1)
                return (gg0, gg1)
            else:
                gg0, gg1 = carry
                gg1 = lax.cond(g0 < ng, lambda gg: group(g0, 1, gg, ssem1),
                               lambda gg: gg, gg1)
                gg0 = lax.cond(g1 < ng, lambda gg: group(g1, 0, gg, ssem0),
                               lambda gg: gg, gg0)
                return (gg0, gg1)

        return lax.fori_loop(0, lax.div(ng + 1, 2), gpair, carry)

    def chunk_pair(tp, carry):
        t0 = tp * 2
        t1 = tp * 2 + 1

        def do0(c):
            wait_fetch(t0, 0, gsem0)
            c = process(t0, 0, c)

            @pl.when(t0 + 2 < nt)
            def _f2():
                fire(t0 + 2, 0, gsem0)
            return c
        carry = lax.cond(t0 < nt, do0, lambda c: c, carry)

        def do1(c):
            wait_fetch(t1, 1, gsem1)
            c = process(t1, 1, c)

            @pl.when(t1 + 2 < nt)
            def _f3():
                fire(t1 + 2, 1, gsem1)
            return c
        carry = lax.cond(t1 < nt, do1, lambda c: c, carry)
        return carry

    gg0, gg1 = lax.fori_loop(0, lax.div(nt + 1, 2), chunk_pair,
                             (jnp.int32(0), jnp.int32(0)))

    @pl.when(gg0 >= 1)
    def _d0():
        pltpu.make_async_copy(comb_v.at[0], out_hbm.at[bidx_v.at[0]], ssem0).wait()

    @pl.when(gg1 >= 1)
    def _d1():
        pltpu.make_async_copy(comb_v.at[1], out_hbm.at[bidx_v.at[1]], ssem1).wait()


def kernel(location_idx, age_idx, W_location, W_age):
    lidx = location_idx.astype(jnp.int32)
    aidx = age_idx.astype(jnp.int32)
    WT = W_location.T
    AT = W_age.T
    tailT = jnp.pad(W_location[_TAIL0:].T,
                    ((0, 0), (0, _RPC - (_NLOC - _TAIL0))))

    mesh = plsc.VectorSubcoreMesh(core_axis_name="c", subcore_axis_name="s")
    run = pl.kernel(
        _body,
        out_type=jax.ShapeDtypeStruct((_BATCH, 2 * _D), jnp.float32),
        mesh=mesh,
        scratch_types=[
            pltpu.VMEM((_BATCH + 16,), jnp.int32),
            pltpu.VMEM((_BATCH + 16,), jnp.int32),
            pltpu.VMEM((_BATCH + 16,), jnp.int32),
            pltpu.VMEM((2, _D, _RPC), jnp.float32),
            pltpu.VMEM((_D, 100), jnp.float32),
            pltpu.VMEM((2, 16, 2 * _D), jnp.float32),
            pltpu.VMEM((2, 16), jnp.int32),
            pltpu.SMEM((_MAXT + 2,), jnp.int32),
            pltpu.SMEM((_MAXT + 2,), jnp.int32),
            pltpu.SemaphoreType.DMA,
            pltpu.SemaphoreType.DMA,
            pltpu.SemaphoreType.DMA,
            pltpu.SemaphoreType.DMA,
        ],
        compiler_params=pltpu.CompilerParams(needs_layout_passes=False),
    )
    return run(lidx, aidx, WT, AT, tailT)


# speedup vs baseline: 5.7605x; 1.0077x over previous
import jax
import jax.numpy as jnp
from jax import lax
from jax.experimental import pallas as pl
from jax.experimental.pallas import tpu as pltpu
from jax.experimental.pallas import tpu_sc as plsc

_BATCH = 16384
_D = 64
_NC = 2
_NS = 16
_NW = _NC * _NS
_NLOC = 1000000
_RPC = 512
_NCHUNKS = (_NLOC + _RPC - 1) // _RPC
_TAILK = _NLOC // _RPC
_TAIL0 = _TAILK * _RPC
_MAXT = (_NCHUNKS - 1) // _NW + 1


def _body(lidx_hbm, aidx_hbm, wt_hbm, at_hbm, tail_hbm, out_hbm,
          lidx_v, aidx_v, mylist_v, chunk_v, atab_v, comb_v, bidx_v,
          hist_s, start_s, gsem0, gsem1, ssem0, ssem1):
    wid = lax.axis_index("s") * _NC + lax.axis_index("c")
    sub = lax.iota(jnp.int32, 16)

    for t0 in range(2):
        pltpu.async_copy(wt_hbm.at[:, pl.ds((wid + t0 * _NW) * _RPC, _RPC)],
                         chunk_v.at[t0], (gsem0, gsem1)[t0])

    pltpu.async_copy(aidx_hbm, aidx_v.at[pl.ds(0, _BATCH)], ssem0)
    pltpu.async_copy(at_hbm, atab_v, ssem0)
    pltpu.sync_copy(lidx_hbm, lidx_v.at[pl.ds(0, _BATCH)])

    def scan_step(v, cnt):
        r = lidx_v[pl.ds(v * 16, 16)]
        k = lax.shift_right_logical(r, 9)
        mine = (k & (_NW - 1)) == wid
        t = lax.shift_right_logical(r, 14)
        x = r & (_RPC - 1)
        b = v * 16 + sub
        packed = (t << 23) | (x << 14) | b
        plsc.store_compressed(mylist_v.at[pl.ds(cnt, 16)], packed, mask=mine)
        return cnt + plsc.all_reduce_population_count(mine)[0]

    n = lax.fori_loop(0, _BATCH // 16, scan_step, jnp.int32(0))

    def zero_step(t, c):
        hist_s[t] = jnp.int32(0)
        return c
    lax.fori_loop(0, _MAXT + 1, zero_step, 0)

    nfull = lax.shift_right_logical(n, 4)
    ntail = n & 15

    def hist_full(v, c):
        pv = mylist_v[pl.ds(v * 16, 16)]
        for j in range(16):
            t = lax.shift_right_logical(pv[j], 23)
            hist_s[t] = hist_s[t] + 1
        return c
    lax.fori_loop(0, nfull, hist_full, 0)

    @pl.when(ntail > 0)
    def _hist_tail():
        pv = mylist_v[pl.ds(nfull * 16, 16)]
        for j in range(16):
            @pl.when(j < ntail)
            def _do():
                t = lax.shift_right_logical(pv[j], 23)
                hist_s[t] = hist_s[t] + 1

    def pfx_step(t, acc):
        c = hist_s[t]
        start_s[t] = acc
        hist_s[t] = acc
        return acc + c
    total = lax.fori_loop(0, _MAXT + 1, pfx_step, jnp.int32(0))
    start_s[_MAXT + 1] = total

    lane0 = sub == 0

    def sort_full(v, c):
        pv = mylist_v[pl.ds(v * 16, 16)]
        for j in range(16):
            t = lax.shift_right_logical(pv[j], 23)
            p = hist_s[t]
            hist_s[t] = p + 1
            idx = jnp.broadcast_to(p, (16,))
            plsc.store_scatter(lidx_v, [idx],
                               jnp.broadcast_to(pv[j], (16,)), mask=lane0)
        return c
    lax.fori_loop(0, nfull, sort_full, 0)

    @pl.when(ntail > 0)
    def _sort_tail():
        pv = mylist_v[pl.ds(nfull * 16, 16)]
        for j in range(16):
            @pl.when(j < ntail)
            def _do():
                t = lax.shift_right_logical(pv[j], 23)
                p = hist_s[t]
                hist_s[t] = p + 1
                idx = jnp.broadcast_to(p, (16,))
                plsc.store_scatter(lidx_v, [idx],
                                   jnp.broadcast_to(pv[j], (16,)), mask=lane0)

    pltpu.make_async_copy(aidx_hbm, aidx_v.at[pl.ds(0, _BATCH)], ssem0).wait()
    pltpu.make_async_copy(at_hbm, atab_v, ssem0).wait()

    nt = lax.div(jnp.int32(_NCHUNKS) - 1 - wid, _NW) + 1

    def m_of(t):
        return start_s[t + 1] - start_s[t]

    def fire(t, slot, gsem):
        k = wid + t * _NW

        @pl.when(m_of(t) > 0)
        def _f():
            @pl.when(k < _TAILK)
            def _a():
                pltpu.async_copy(wt_hbm.at[:, pl.ds(k * _RPC, _RPC)],
                                 chunk_v.at[slot], gsem)

            @pl.when(k == _TAILK)
            def _b():
                pltpu.async_copy(tail_hbm, chunk_v.at[slot], gsem)

    def wait_fetch(t, slot, gsem):
        @pl.when((m_of(t) > 0) | (t < 2))
        def _w():
            pltpu.make_async_copy(tail_hbm, chunk_v.at[slot], gsem).wait()

    def process(t, cslot, carry):
        s0 = start_s[t]
        m = m_of(t)
        ng = lax.div(m + 15, 16)

        def group(g, gpar, gg, ssem):
            pv = lidx_v[pl.ds(s0 + g * 16, 16)]
            rem = m - g * 16
            bvec = pv & jnp.int32(16383)
            bvec = jnp.where(sub < rem, bvec, jnp.broadcast_to(bvec[0], (16,)))

            @pl.when(gg >= 1)
            def _wprev():
                pltpu.make_async_copy(comb_v.at[gpar],
                                      out_hbm.at[bidx_v.at[gpar]], ssem).wait()

            for j in range(16):
                @pl.when(j < rem)
                def _fill():
                    x = lax.shift_right_logical(pv[j], 14) & (_RPC - 1)
                    a = aidx_v[pl.ds(pv[j] & 16383, 16)][0]
                    for c in range(_D // 16):
                        d16 = sub + c * 16
                        ag = plsc.load_gather(
                            atab_v, [d16, jnp.broadcast_to(a, (16,))])
                        comb_v[gpar, j, pl.ds(c * 16, 16)] = ag
                        lg = plsc.load_gather(
                            chunk_v.at[cslot],
                            [d16, jnp.broadcast_to(x, (16,))])
                        comb_v[gpar, j, pl.ds(_D + c * 16, 16)] = lg

                @pl.when(j >= rem)
                def _pad():
                    for c in range(2 * _D // 16):
                        comb_v[gpar, j, pl.ds(c * 16, 16)] = \
                            comb_v[gpar, 0, pl.ds(c * 16, 16)]

            bidx_v[gpar, :] = bvec
            pltpu.async_copy(comb_v.at[gpar], out_hbm.at[bidx_v.at[gpar]], ssem)
            return gg + 1

        def gpair(gp, carry):
            g0 = gp * 2
            g1 = gp * 2 + 1
            if cslot == 0:
                gg0, gg1 = carry
                gg0 = lax.cond(g0 < ng, lambda gg: group(g0, 0, gg, ssem0),
                               lambda gg: gg, gg0)
                gg1 = lax.cond(g1 < ng, lambda gg: group(g1, 1, gg, ssem1),
                               lambda gg: gg, gg1)
                return (gg0, gg1)
            else:
                gg0, gg1 = carry
                gg1 = lax.cond(g0 < ng, lambda gg: group(g0, 1, gg, ssem1),
                               lambda gg: gg, gg1)
                gg0 = lax.cond(g1 < ng, lambda gg: group(g1, 0, gg, ssem0),
                               lambda gg: gg, gg0)
                return (gg0, gg1)

        return lax.fori_loop(0, lax.div(ng + 1, 2), gpair, carry)

    def chunk_pair(tp, carry):
        t0 = tp * 2
        t1 = tp * 2 + 1

        def do0(c):
            wait_fetch(t0, 0, gsem0)
            c = process(t0, 0, c)

            @pl.when(t0 + 2 < nt)
            def _f2():
                fire(t0 + 2, 0, gsem0)
            return c
        carry = lax.cond(t0 < nt, do0, lambda c: c, carry)

        def do1(c):
            wait_fetch(t1, 1, gsem1)
            c = process(t1, 1, c)

            @pl.when(t1 + 2 < nt)
            def _f3():
                fire(t1 + 2, 1, gsem1)
            return c
        carry = lax.cond(t1 < nt, do1, lambda c: c, carry)
        return carry

    gg0, gg1 = lax.fori_loop(0, lax.div(nt + 1, 2), chunk_pair,
                             (jnp.int32(0), jnp.int32(0)))

    @pl.when(gg0 >= 1)
    def _d0():
        pltpu.make_async_copy(comb_v.at[0], out_hbm.at[bidx_v.at[0]], ssem0).wait()

    @pl.when(gg1 >= 1)
    def _d1():
        pltpu.make_async_copy(comb_v.at[1], out_hbm.at[bidx_v.at[1]], ssem1).wait()


def kernel(location_idx, age_idx, W_location, W_age):
    lidx = location_idx.astype(jnp.int32)
    aidx = age_idx.astype(jnp.int32)
    WT = W_location.T
    AT = W_age.T
    tailT = jnp.pad(W_location[_TAIL0:].T,
                    ((0, 0), (0, _RPC - (_NLOC - _TAIL0))))

    mesh = plsc.VectorSubcoreMesh(core_axis_name="c", subcore_axis_name="s")
    run = pl.kernel(
        _body,
        out_type=jax.ShapeDtypeStruct((_BATCH, 2 * _D), jnp.float32),
        mesh=mesh,
        scratch_types=[
            pltpu.VMEM((_BATCH + 16,), jnp.int32),
            pltpu.VMEM((_BATCH + 16,), jnp.int32),
            pltpu.VMEM((_BATCH + 16,), jnp.int32),
            pltpu.VMEM((2, _D, _RPC), jnp.float32),
            pltpu.VMEM((_D, 100), jnp.float32),
            pltpu.VMEM((2, 16, 2 * _D), jnp.float32),
            pltpu.VMEM((2, 16), jnp.int32),
            pltpu.SMEM((_MAXT + 2,), jnp.int32),
            pltpu.SMEM((_MAXT + 2,), jnp.int32),
            pltpu.SemaphoreType.DMA,
            pltpu.SemaphoreType.DMA,
            pltpu.SemaphoreType.DMA,
            pltpu.SemaphoreType.DMA,
        ],
        compiler_params=pltpu.CompilerParams(needs_layout_passes=False),
    )
    return run(lidx, aidx, WT, AT, tailT)
